# baseline (speedup 1.0000x reference)
import functools

import jax
import jax.numpy as jnp
from jax import lax
from jax.experimental import pallas as pl
from jax.experimental.pallas import tpu as pltpu
from jax.experimental.pallas import tpu_sc as plsc

NNZ = 320000
NSEG = 10000
NSEG_PAD = 10240
DIM = 128
CHUNK = 128
NCHUNKS = NNZ // CHUNK
NC = 2
NS = 16
NW = NC * NS
SEG_PER_TILE = NSEG_PAD // NS
ZROWS = 128

_f32 = jnp.float32
_i32 = jnp.int32


def _sc_mesh():
  return plsc.VectorSubcoreMesh(core_axis_name="c", subcore_axis_name="s",
                                num_cores=NC, num_subcores=NS)


def _pools_body(values_hbm, rows_hbm, cols_hbm,
                row_sums_hbm, col_sums_hbm, rowcnt_hbm, colcnt_hbm,
                table, vbuf0, vbuf1, ibuf0, ibuf1,
                sem_li0, sem_li1, sem_lv0, sem_lv1, sem_s0, sem_s1):
  c = lax.axis_index("c")
  s = lax.axis_index("s")
  n_iters = (NCHUNKS + NS - 1) // NS
  vbufs = (vbuf0, vbuf1)
  ibufs = (ibuf0, ibuf1)
  sem_li = (sem_li0, sem_li1)
  sem_lv = (sem_lv0, sem_lv1)
  sem_s = (sem_s0, sem_s1)

  def fill_vbuf(b, val):
    def fill_row(i, _):
      for j in range(DIM // 16):
        vbufs[b][i, pl.ds(j * 16, 16)] = jnp.full((16,), val, _f32)
      return _
    lax.fori_loop(0, CHUNK, fill_row, None)

  def zero_table():
    for k in range(SEG_PER_TILE // ZROWS):
      off = pl.multiple_of(s * SEG_PER_TILE + k * ZROWS, ZROWS)
      pltpu.sync_copy(vbuf0, table.at[pl.ds(off, ZROWS)])

  def chunk_of(t):
    return s + t * NS

  def base_of(t):
    return pl.multiple_of(chunk_of(t) * CHUNK, CHUNK)

  def scatter_pass(idx_hbm, with_values):
    def fire_loads(b, t):
      base = base_of(t)
      pltpu.async_copy(idx_hbm.at[pl.ds(base, CHUNK)], ibufs[b], sem_li[b])
      if with_values:
        pltpu.async_copy(values_hbm.at[pl.ds(base, CHUNK)], vbufs[b],
                         sem_lv[b])

    def wait_loads(b, t):
      base = base_of(t)
      pltpu.make_async_copy(idx_hbm.at[pl.ds(base, CHUNK)], ibufs[b],
                            sem_li[b]).wait()
      if with_values:
        pltpu.make_async_copy(values_hbm.at[pl.ds(base, CHUNK)], vbufs[b],
                              sem_lv[b]).wait()

    def wait_scatter(b):
      src = vbufs[b] if with_values else vbuf0
      pltpu.make_async_copy(src, table.at[ibufs[b]], sem_s[b]).wait()

    for b in range(2):
      @pl.when(chunk_of(b) < NCHUNKS)
      def _(b=b):
        fire_loads(b, b)

    def body(t2, _):
      t0 = t2 * 2
      for b in range(2):
        t = t0 + b

        @pl.when(chunk_of(t) < NCHUNKS)
        def _(b=b, t=t):
          wait_loads(b, t)
          src = vbufs[b] if with_values else vbuf0
          pltpu.async_copy(src, table.at[ibufs[b]], sem_s[b], add=True)
      for b in range(2):
        t = t0 + b

        @pl.when(chunk_of(t + 2) < NCHUNKS)
        def _(b=b, t=t):
          wait_scatter(b)
          fire_loads(b, t + 2)
      return _
    lax.fori_loop(0, (n_iters + 1) // 2, body, None)

    for db in range(3):
      t_tail = n_iters - 3 + db

      @pl.when((chunk_of(t_tail) < NCHUNKS)
               & (chunk_of(t_tail + 2) >= NCHUNKS))
      def _(b=t_tail % 2):
        wait_scatter(b)

  def writeback(dst_hbm):
    seg0 = pl.multiple_of(s * SEG_PER_TILE, ZROWS)
    pltpu.sync_copy(table.at[pl.ds(seg0, SEG_PER_TILE)],
                    dst_hbm.at[pl.ds(seg0, SEG_PER_TILE)])

  fill_vbuf(0, 0.0)
  zero_table()
  plsc.subcore_barrier()

  @pl.when(c == 0)
  def _():
    scatter_pass(rows_hbm, True)

  @pl.when(c == 1)
  def _():
    scatter_pass(cols_hbm, True)

  plsc.subcore_barrier()

  @pl.when(c == 0)
  def _():
    writeback(row_sums_hbm)

  @pl.when(c == 1)
  def _():
    writeback(col_sums_hbm)

  fill_vbuf(0, 0.0)
  zero_table()
  plsc.subcore_barrier()
  fill_vbuf(0, 1.0)

  @pl.when(c == 0)
  def _():
    scatter_pass(rows_hbm, False)

  @pl.when(c == 1)
  def _():
    scatter_pass(cols_hbm, False)

  plsc.subcore_barrier()

  @pl.when(c == 0)
  def _():
    writeback(rowcnt_hbm)

  @pl.when(c == 1)
  def _():
    writeback(colcnt_hbm)


@jax.jit
def _pools(values, rows, cols):
  f = functools.partial(
      pl.kernel,
      out_type=(
          jax.ShapeDtypeStruct((NSEG_PAD, DIM), _f32),
          jax.ShapeDtypeStruct((NSEG_PAD, DIM), _f32),
          jax.ShapeDtypeStruct((NSEG_PAD, DIM), _f32),
          jax.ShapeDtypeStruct((NSEG_PAD, DIM), _f32),
      ),
      mesh=_sc_mesh(),
      scratch_types=[
          pltpu.VMEM_SHARED((NSEG_PAD, DIM), _f32),
          pltpu.VMEM((CHUNK, DIM), _f32),
          pltpu.VMEM((CHUNK, DIM), _f32),
          pltpu.VMEM((CHUNK,), _i32),
          pltpu.VMEM((CHUNK,), _i32),
          pltpu.SemaphoreType.DMA,
          pltpu.SemaphoreType.DMA,
          pltpu.SemaphoreType.DMA,
          pltpu.SemaphoreType.DMA,
          pltpu.SemaphoreType.DMA,
          pltpu.SemaphoreType.DMA,
      ],
  )(_pools_body)
  return f(values, rows, cols)


def _tables_body(rs_ref, cs_ref, rc_ref, cc_ref, w_ref, rbc_ref, cbc_ref):
  rs = rs_ref[...]
  cs = cs_ref[...]
  rc = rc_ref[...][:, :1]
  cc = cc_ref[...][:, :1]
  w1 = w_ref[1]
  w2 = w_ref[2]
  w3 = w_ref[3]
  rp = rs * (1.0 / jnp.maximum(rc, 1.0))
  cp = cs * (1.0 / jnp.maximum(cc, 1.0))
  all_vec = (jnp.sum(rs, axis=0, keepdims=True) / NNZ)
  rbc_ref[...] = (jnp.dot(rp, w1, preferred_element_type=_f32)
                  + jnp.dot(all_vec, w3, preferred_element_type=_f32))
  cbc_ref[...] = jnp.dot(cp, w2, preferred_element_type=_f32)


@jax.jit
def _tables(row_sums, col_sums, row_cnt, col_cnt, weights):
  return pl.pallas_call(
      _tables_body,
      out_shape=(
          jax.ShapeDtypeStruct((NSEG_PAD, DIM), _f32),
          jax.ShapeDtypeStruct((NSEG_PAD, DIM), _f32),
      ),
  )(row_sums, col_sums, row_cnt, col_cnt, weights)


def _gather_body(rows_hbm, cols_hbm, rbc_hbm, cbc_hbm, g_hbm,
                 rbuf0, rbuf1, rbuf2, cbuf0, cbuf1, cbuf2,
                 ribuf0, ribuf1, ribuf2, cibuf0, cibuf1, cibuf2,
                 sem_ri0, sem_ri1, sem_ri2, sem_ci0, sem_ci1, sem_ci2,
                 sem_gr0, sem_gr1, sem_gr2, sem_gc0, sem_gc1, sem_gc2,
                 sem_o0, sem_o1, sem_o2):
  c = lax.axis_index("c")
  s = lax.axis_index("s")
  wid = s * NC + c
  n_iters = (NCHUNKS + NW - 1) // NW
  rbufs = (rbuf0, rbuf1, rbuf2)
  cbufs = (cbuf0, cbuf1, cbuf2)
  ribufs = (ribuf0, ribuf1, ribuf2)
  cibufs = (cibuf0, cibuf1, cibuf2)
  sem_ri = (sem_ri0, sem_ri1, sem_ri2)
  sem_ci = (sem_ci0, sem_ci1, sem_ci2)
  sem_gr = (sem_gr0, sem_gr1, sem_gr2)
  sem_gc = (sem_gc0, sem_gc1, sem_gc2)
  sem_o = (sem_o0, sem_o1, sem_o2)

  def chunk_of(t):
    return wid + t * NW

  def base_of(t):
    return pl.multiple_of(chunk_of(t) * CHUNK, CHUNK)

  def fire_idx(b, t):
    base = base_of(t)
    pltpu.async_copy(rows_hbm.at[pl.ds(base, CHUNK)], ribufs[b], sem_ri[b])
    pltpu.async_copy(cols_hbm.at[pl.ds(base, CHUNK)], cibufs[b], sem_ci[b])

  def wait_idx(b, t):
    base = base_of(t)
    pltpu.make_async_copy(rows_hbm.at[pl.ds(base, CHUNK)], ribufs[b],
                          sem_ri[b]).wait()
    pltpu.make_async_copy(cols_hbm.at[pl.ds(base, CHUNK)], cibufs[b],
                          sem_ci[b]).wait()

  def fire_gathers(b):
    pltpu.async_copy(rbc_hbm.at[ribufs[b]], rbufs[b], sem_gr[b])
    pltpu.async_copy(cbc_hbm.at[cibufs[b]], cbufs[b], sem_gc[b])

  def wait_gathers(b):
    pltpu.make_async_copy(rbc_hbm.at[ribufs[b]], rbufs[b], sem_gr[b]).wait()
    pltpu.make_async_copy(cbc_hbm.at[cibufs[b]], cbufs[b], sem_gc[b]).wait()

  def wait_store(b, t):
    base = base_of(t)
    pltpu.make_async_copy(rbufs[b], g_hbm.at[pl.ds(base, CHUNK)],
                          sem_o[b]).wait()

  for b in range(2):
    @pl.when(chunk_of(b) < NCHUNKS)
    def _(b=b):
      fire_idx(b, b)

  @pl.when(chunk_of(0) < NCHUNKS)
  def _():
    wait_idx(0, 0)
    fire_gathers(0)

  def body(t3, _):
    tbase = t3 * 3
    for b3 in range(3):
      t = tbase + b3
      sl_t = b3
      sl_n = (b3 + 1) % 3
      sl_p = (b3 + 2) % 3

      @pl.when(chunk_of(t + 1) < NCHUNKS)
      def _(t=t, sl_n=sl_n):
        pltpu.make_async_copy(
            rows_hbm.at[pl.ds(base_of(t + 1), CHUNK)], ribufs[sl_n],
            sem_ri[sl_n]).wait()
        pltpu.make_async_copy(
            cols_hbm.at[pl.ds(base_of(t + 1), CHUNK)], cibufs[sl_n],
            sem_ci[sl_n]).wait()

        @pl.when((t >= 2) & (chunk_of(t - 2) < NCHUNKS))
        def _():
          wait_store(sl_n, t - 2)
        fire_gathers(sl_n)

      @pl.when(chunk_of(t) < NCHUNKS)
      def _(t=t, sl_t=sl_t):
        wait_gathers(sl_t)

        def add_row(i2, carry):
          for u in range(4):
            i = i2 * 4 + u
            for q in range(DIM // 16):
              sl = pl.ds(q * 16, 16)
              rbufs[sl_t][i, sl] = rbufs[sl_t][i, sl] + cbufs[sl_t][i, sl]
          return carry
        lax.fori_loop(0, CHUNK // 4, add_row, None)
        pltpu.async_copy(rbufs[sl_t], g_hbm.at[pl.ds(base_of(t), CHUNK)],
                         sem_o[sl_t])

      @pl.when(chunk_of(t + 2) < NCHUNKS)
      def _(t=t, sl_p=sl_p):
        fire_idx(sl_p, t + 2)
    return _
  lax.fori_loop(0, (n_iters + 2) // 3, body, None)

  for db in range(4):
    t_tail = n_iters - 4 + db

    @pl.when((chunk_of(t_tail) < NCHUNKS)
             & (chunk_of(t_tail + 3) >= NCHUNKS))
    def _(t_tail=t_tail):
      wait_store(t_tail % 3, t_tail)


@jax.jit
def _gather(rows, cols, row_bc, col_bc):
  f = functools.partial(
      pl.kernel,
      out_type=jax.ShapeDtypeStruct((NNZ, DIM), _f32),
      mesh=_sc_mesh(),
      scratch_types=(
          [pltpu.VMEM((CHUNK, DIM), _f32)] * 6
          + [pltpu.VMEM((CHUNK,), _i32)] * 6
          + [pltpu.SemaphoreType.DMA] * 15
      ),
  )(_gather_body)
  return f(rows, cols, row_bc, col_bc)


_MM_BLOCK = 3200


def _matmul_add_body(v_ref, w_ref, g_ref, o_ref):
  o_ref[...] = (jnp.dot(v_ref[...], w_ref[...], preferred_element_type=_f32)
                + g_ref[...])


@jax.jit
def _matmul_add(values, w0, g):
  return pl.pallas_call(
      _matmul_add_body,
      grid=(NNZ // _MM_BLOCK,),
      in_specs=[
          pl.BlockSpec((_MM_BLOCK, DIM), lambda b: (b, 0)),
          pl.BlockSpec((DIM, DIM), lambda b: (0, 0)),
          pl.BlockSpec((_MM_BLOCK, DIM), lambda b: (b, 0)),
      ],
      out_specs=pl.BlockSpec((_MM_BLOCK, DIM), lambda b: (b, 0)),
      out_shape=jax.ShapeDtypeStruct((NNZ, DIM), _f32),
  )(values, w0, g)


def kernel(values, indices, weights):
  idx = indices.astype(_i32)
  rows = idx[0]
  cols = idx[1]
  row_sums, col_sums, row_cnt, col_cnt = _pools(values, rows, cols)
  row_bc, col_bc = _tables(row_sums, col_sums, row_cnt, col_cnt, weights)
  g = _gather(rows, cols, row_bc, col_bc)
  return _matmul_add(values, weights[0], g)


# speedup vs baseline: 1.1464x; 1.1464x over previous
import functools

import jax
import jax.numpy as jnp
from jax import lax
from jax.experimental import pallas as pl
from jax.experimental.pallas import tpu as pltpu
from jax.experimental.pallas import tpu_sc as plsc

NNZ = 320000
NSEG = 10000
NSEG_PAD = 10240
DIM = 128
CHUNK = 80
NCHUNKS = NNZ // CHUNK
NC = 2
NS = 16
NW = NC * NS
SEG_PER_TILE = NSEG_PAD // NS
ZROWS = 80

_f32 = jnp.float32
_i32 = jnp.int32


def _sc_mesh():
  return plsc.VectorSubcoreMesh(core_axis_name="c", subcore_axis_name="s",
                                num_cores=NC, num_subcores=NS)


def _pools_body(values_hbm, rows_hbm, cols_hbm,
                row_sums_hbm, col_sums_hbm, rowcnt_hbm, colcnt_hbm,
                table, vbuf0, vbuf1, vbuf2, ibuf0, ibuf1, ibuf2,
                sem_li0, sem_li1, sem_li2, sem_lv0, sem_lv1, sem_lv2,
                sem_s0, sem_s1, sem_s2):
  c = lax.axis_index("c")
  s = lax.axis_index("s")
  n_iters = (NCHUNKS + NS - 1) // NS
  vbufs = (vbuf0, vbuf1, vbuf2)
  ibufs = (ibuf0, ibuf1, ibuf2)
  sem_li = (sem_li0, sem_li1, sem_li2)
  sem_lv = (sem_lv0, sem_lv1, sem_lv2)
  sem_s = (sem_s0, sem_s1, sem_s2)

  def fill_vbuf(b, val):
    def fill_row(i, _):
      for j in range(DIM // 16):
        vbufs[b][i, pl.ds(j * 16, 16)] = jnp.full((16,), val, _f32)
      return _
    lax.fori_loop(0, CHUNK, fill_row, None)

  def zero_table():
    for k in range(SEG_PER_TILE // ZROWS):
      off = pl.multiple_of(s * SEG_PER_TILE + k * ZROWS, ZROWS)
      pltpu.sync_copy(vbuf0, table.at[pl.ds(off, ZROWS)])

  def chunk_of(t):
    return s + t * NS

  def base_of(t):
    return pl.multiple_of(chunk_of(t) * CHUNK, CHUNK)

  def scatter_pass(idx_hbm, with_values):
    def fire_loads(b, t):
      base = base_of(t)
      pltpu.async_copy(idx_hbm.at[pl.ds(base, CHUNK)], ibufs[b], sem_li[b])
      if with_values:
        pltpu.async_copy(values_hbm.at[pl.ds(base, CHUNK)], vbufs[b],
                         sem_lv[b])

    def wait_loads(b, t):
      base = base_of(t)
      pltpu.make_async_copy(idx_hbm.at[pl.ds(base, CHUNK)], ibufs[b],
                            sem_li[b]).wait()
      if with_values:
        pltpu.make_async_copy(values_hbm.at[pl.ds(base, CHUNK)], vbufs[b],
                              sem_lv[b]).wait()

    def wait_scatter(b):
      src = vbufs[b] if with_values else vbuf0
      pltpu.make_async_copy(src, table.at[ibufs[b]], sem_s[b]).wait()

    for b in range(2):
      @pl.when(chunk_of(b) < NCHUNKS)
      def _(b=b):
        fire_loads(b, b)

    def body(t3, _):
      tbase = t3 * 3
      for b3 in range(3):
        t = tbase + b3
        sl_t = b3
        sl_p = (b3 + 2) % 3

        @pl.when(chunk_of(t) < NCHUNKS)
        def _(t=t, sl_t=sl_t):
          wait_loads(sl_t, t)
          src = vbufs[sl_t] if with_values else vbuf0
          pltpu.async_copy(src, table.at[ibufs[sl_t]], sem_s[sl_t],
                           add=True)

        @pl.when(chunk_of(t + 2) < NCHUNKS)
        def _(t=t, sl_p=sl_p):
          @pl.when((t >= 1) & (chunk_of(t - 1) < NCHUNKS))
          def _():
            wait_scatter(sl_p)
          fire_loads(sl_p, t + 2)
      return _
    lax.fori_loop(0, (n_iters + 2) // 3, body, None)

    for db in range(4):
      t_tail = n_iters - 4 + db

      @pl.when((chunk_of(t_tail) < NCHUNKS)
               & (chunk_of(t_tail + 3) >= NCHUNKS))
      def _(t_tail=t_tail):
        wait_scatter(t_tail % 3)

  def writeback(dst_hbm):
    seg0 = pl.multiple_of(s * SEG_PER_TILE, ZROWS)
    pltpu.sync_copy(table.at[pl.ds(seg0, SEG_PER_TILE)],
                    dst_hbm.at[pl.ds(seg0, SEG_PER_TILE)])

  fill_vbuf(0, 0.0)
  zero_table()
  plsc.subcore_barrier()

  @pl.when(c == 0)
  def _():
    scatter_pass(rows_hbm, True)

  @pl.when(c == 1)
  def _():
    scatter_pass(cols_hbm, True)

  plsc.subcore_barrier()

  @pl.when(c == 0)
  def _():
    writeback(row_sums_hbm)

  @pl.when(c == 1)
  def _():
    writeback(col_sums_hbm)

  fill_vbuf(0, 0.0)
  zero_table()
  plsc.subcore_barrier()
  fill_vbuf(0, 1.0)

  @pl.when(c == 0)
  def _():
    scatter_pass(rows_hbm, False)

  @pl.when(c == 1)
  def _():
    scatter_pass(cols_hbm, False)

  plsc.subcore_barrier()

  @pl.when(c == 0)
  def _():
    writeback(rowcnt_hbm)

  @pl.when(c == 1)
  def _():
    writeback(colcnt_hbm)


@jax.jit
def _pools(values, rows, cols):
  f = functools.partial(
      pl.kernel,
      out_type=(
          jax.ShapeDtypeStruct((NSEG_PAD, DIM), _f32),
          jax.ShapeDtypeStruct((NSEG_PAD, DIM), _f32),
          jax.ShapeDtypeStruct((NSEG_PAD, DIM), _f32),
          jax.ShapeDtypeStruct((NSEG_PAD, DIM), _f32),
      ),
      mesh=_sc_mesh(),
      scratch_types=(
          [pltpu.VMEM_SHARED((NSEG_PAD, DIM), _f32)]
          + [pltpu.VMEM((CHUNK, DIM), _f32)] * 3
          + [pltpu.VMEM((CHUNK,), _i32)] * 3
          + [pltpu.SemaphoreType.DMA] * 9
      ),
  )(_pools_body)
  return f(values, rows, cols)


def _tables_body(rs_ref, cs_ref, rc_ref, cc_ref, w_ref, rbc_ref, cbc_ref):
  rs = rs_ref[...]
  cs = cs_ref[...]
  rc = rc_ref[...][:, :1]
  cc = cc_ref[...][:, :1]
  w1 = w_ref[1]
  w2 = w_ref[2]
  w3 = w_ref[3]
  rp = rs * (1.0 / jnp.maximum(rc, 1.0))
  cp = cs * (1.0 / jnp.maximum(cc, 1.0))
  all_vec = (jnp.sum(rs, axis=0, keepdims=True) / NNZ)
  rbc_ref[...] = (jnp.dot(rp, w1, preferred_element_type=_f32)
                  + jnp.dot(all_vec, w3, preferred_element_type=_f32))
  cbc_ref[...] = jnp.dot(cp, w2, preferred_element_type=_f32)


@jax.jit
def _tables(row_sums, col_sums, row_cnt, col_cnt, weights):
  return pl.pallas_call(
      _tables_body,
      out_shape=(
          jax.ShapeDtypeStruct((NSEG_PAD, DIM), _f32),
          jax.ShapeDtypeStruct((NSEG_PAD, DIM), _f32),
      ),
  )(row_sums, col_sums, row_cnt, col_cnt, weights)


def _gather_body(rows_hbm, cols_hbm, rbc_hbm, cbc_hbm, g_hbm,
                 rbuf0, rbuf1, rbuf2, cbuf0, cbuf1, cbuf2,
                 ribuf0, ribuf1, ribuf2, cibuf0, cibuf1, cibuf2,
                 sem_ri0, sem_ri1, sem_ri2, sem_ci0, sem_ci1, sem_ci2,
                 sem_gr0, sem_gr1, sem_gr2, sem_gc0, sem_gc1, sem_gc2,
                 sem_o0, sem_o1, sem_o2):
  c = lax.axis_index("c")
  s = lax.axis_index("s")
  wid = s * NC + c
  n_iters = (NCHUNKS + NW - 1) // NW
  rbufs = (rbuf0, rbuf1, rbuf2)
  cbufs = (cbuf0, cbuf1, cbuf2)
  ribufs = (ribuf0, ribuf1, ribuf2)
  cibufs = (cibuf0, cibuf1, cibuf2)
  sem_ri = (sem_ri0, sem_ri1, sem_ri2)
  sem_ci = (sem_ci0, sem_ci1, sem_ci2)
  sem_gr = (sem_gr0, sem_gr1, sem_gr2)
  sem_gc = (sem_gc0, sem_gc1, sem_gc2)
  sem_o = (sem_o0, sem_o1, sem_o2)

  def chunk_of(t):
    return wid + t * NW

  def base_of(t):
    return pl.multiple_of(chunk_of(t) * CHUNK, CHUNK)

  def fire_idx(b, t):
    base = base_of(t)
    pltpu.async_copy(rows_hbm.at[pl.ds(base, CHUNK)], ribufs[b], sem_ri[b])
    pltpu.async_copy(cols_hbm.at[pl.ds(base, CHUNK)], cibufs[b], sem_ci[b])

  def wait_idx(b, t):
    base = base_of(t)
    pltpu.make_async_copy(rows_hbm.at[pl.ds(base, CHUNK)], ribufs[b],
                          sem_ri[b]).wait()
    pltpu.make_async_copy(cols_hbm.at[pl.ds(base, CHUNK)], cibufs[b],
                          sem_ci[b]).wait()

  def fire_gathers(b):
    pltpu.async_copy(rbc_hbm.at[ribufs[b]], rbufs[b], sem_gr[b])
    pltpu.async_copy(cbc_hbm.at[cibufs[b]], cbufs[b], sem_gc[b])

  def wait_gathers(b):
    pltpu.make_async_copy(rbc_hbm.at[ribufs[b]], rbufs[b], sem_gr[b]).wait()
    pltpu.make_async_copy(cbc_hbm.at[cibufs[b]], cbufs[b], sem_gc[b]).wait()

  def wait_store(b, t):
    base = base_of(t)
    pltpu.make_async_copy(rbufs[b], g_hbm.at[pl.ds(base, CHUNK)],
                          sem_o[b]).wait()

  for b in range(2):
    @pl.when(chunk_of(b) < NCHUNKS)
    def _(b=b):
      fire_idx(b, b)

  @pl.when(chunk_of(0) < NCHUNKS)
  def _():
    wait_idx(0, 0)
    fire_gathers(0)

  def body(t3, _):
    tbase = t3 * 3
    for b3 in range(3):
      t = tbase + b3
      sl_t = b3
      sl_n = (b3 + 1) % 3
      sl_p = (b3 + 2) % 3

      @pl.when(chunk_of(t + 1) < NCHUNKS)
      def _(t=t, sl_n=sl_n):
        pltpu.make_async_copy(
            rows_hbm.at[pl.ds(base_of(t + 1), CHUNK)], ribufs[sl_n],
            sem_ri[sl_n]).wait()
        pltpu.make_async_copy(
            cols_hbm.at[pl.ds(base_of(t + 1), CHUNK)], cibufs[sl_n],
            sem_ci[sl_n]).wait()

        @pl.when((t >= 2) & (chunk_of(t - 2) < NCHUNKS))
        def _():
          wait_store(sl_n, t - 2)
        fire_gathers(sl_n)

      @pl.when(chunk_of(t) < NCHUNKS)
      def _(t=t, sl_t=sl_t):
        wait_gathers(sl_t)

        def add_row(i, carry):
          for q in range(DIM // 16):
            sl = pl.ds(q * 16, 16)
            rbufs[sl_t][i, sl] = rbufs[sl_t][i, sl] + cbufs[sl_t][i, sl]
          return carry
        lax.fori_loop(0, CHUNK, add_row, None)
        pltpu.async_copy(rbufs[sl_t], g_hbm.at[pl.ds(base_of(t), CHUNK)],
                         sem_o[sl_t])

      @pl.when(chunk_of(t + 2) < NCHUNKS)
      def _(t=t, sl_p=sl_p):
        fire_idx(sl_p, t + 2)
    return _
  lax.fori_loop(0, (n_iters + 2) // 3, body, None)

  for db in range(4):
    t_tail = n_iters - 4 + db

    @pl.when((chunk_of(t_tail) < NCHUNKS)
             & (chunk_of(t_tail + 3) >= NCHUNKS))
    def _(t_tail=t_tail):
      wait_store(t_tail % 3, t_tail)


@jax.jit
def _gather(rows, cols, row_bc, col_bc):
  f = functools.partial(
      pl.kernel,
      out_type=jax.ShapeDtypeStruct((NNZ, DIM), _f32),
      mesh=_sc_mesh(),
      scratch_types=(
          [pltpu.VMEM((CHUNK, DIM), _f32)] * 6
          + [pltpu.VMEM((CHUNK,), _i32)] * 6
          + [pltpu.SemaphoreType.DMA] * 15
      ),
  )(_gather_body)
  return f(rows, cols, row_bc, col_bc)


_MM_BLOCK = 3200


def _matmul_add_body(v_ref, w_ref, g_ref, o_ref):
  o_ref[...] = (jnp.dot(v_ref[...], w_ref[...], preferred_element_type=_f32)
                + g_ref[...])


@jax.jit
def _matmul_add(values, w0, g):
  return pl.pallas_call(
      _matmul_add_body,
      grid=(NNZ // _MM_BLOCK,),
      in_specs=[
          pl.BlockSpec((_MM_BLOCK, DIM), lambda b: (b, 0)),
          pl.BlockSpec((DIM, DIM), lambda b: (0, 0)),
          pl.BlockSpec((_MM_BLOCK, DIM), lambda b: (b, 0)),
      ],
      out_specs=pl.BlockSpec((_MM_BLOCK, DIM), lambda b: (b, 0)),
      out_shape=jax.ShapeDtypeStruct((NNZ, DIM), _f32),
  )(values, w0, g)


def kernel(values, indices, weights):
  idx = indices.astype(_i32)
  rows = idx[0]
  cols = idx[1]
  row_sums, col_sums, row_cnt, col_cnt = _pools(values, rows, cols)
  row_bc, col_bc = _tables(row_sums, col_sums, row_cnt, col_cnt, weights)
  g = _gather(rows, cols, row_bc, col_bc)
  return _matmul_add(values, weights[0], g)
